# baseline (speedup 1.0000x reference)
import functools
import math

import jax
import jax.numpy as jnp
from jax import lax
from jax.experimental import pallas as pl
from jax.experimental.pallas import tpu as pltpu
from jax.experimental.pallas import tpu_sc as plsc

D_MODEL = 64
D_PAD = 128
SCALE = math.sqrt(D_MODEL)

NUM_CORES = 2
NUM_SUBCORES = 16
NUM_WORKERS = NUM_CORES * NUM_SUBCORES
LANES = 16
NBUF = 5
LOOKAHEAD = 2


@functools.partial(jax.jit, static_argnames=("total_rows", "chunk"))
def _embed(x_flat, table_c, *, total_rows, chunk):
    rows_per_worker = total_rows // NUM_WORKERS
    num_chunks = rows_per_worker // chunk
    assert num_chunks % NBUF == 0
    outer = num_chunks // NBUF
    mesh = plsc.VectorSubcoreMesh(core_axis_name="c", subcore_axis_name="s")

    @functools.partial(
        pl.kernel,
        mesh=mesh,
        out_type=jax.ShapeDtypeStruct((total_rows, D_PAD), jnp.float32),
        scratch_types=[
            pltpu.VMEM((rows_per_worker,), jnp.int32),
            pltpu.VMEM((NBUF, chunk, D_MODEL), jnp.float32),
            [pltpu.SemaphoreType.DMA] * NBUF,
            [pltpu.SemaphoreType.DMA] * NBUF,
        ],
        compiler_params=pltpu.CompilerParams(use_tc_tiling_on_sc=False),
    )
    def k(table_hbm, idx_hbm, out_hbm, idx_v, rows_v, gsems, ssems):
        wid = lax.axis_index("s") * NUM_CORES + lax.axis_index("c")
        base = wid * rows_per_worker
        pltpu.sync_copy(idx_hbm.at[pl.ds(base, rows_per_worker)], idx_v)

        def gather(i, b, sem):
            idx_sl = idx_v.at[pl.ds(i * chunk, chunk)]
            return pltpu.make_async_copy(table_hbm.at[idx_sl], rows_v.at[b], sem)

        def scatter(i, b, sem):
            dst = out_hbm.at[pl.ds(base + i * chunk, chunk), pl.ds(0, D_MODEL)]
            return pltpu.make_async_copy(rows_v.at[b], dst, sem)

        for b in range(LOOKAHEAD):
            gather(b, b, gsems[b]).start()

        def outer_body(t, _):
            for b in range(NBUF):
                i = t * NBUF + b
                j = i + LOOKAHEAD
                bj = (b + LOOKAHEAD) % NBUF

                @pl.when(j < num_chunks)
                def _():
                    @pl.when(j >= NBUF)
                    def _():
                        scatter(j - NBUF, bj, ssems[bj]).wait()

                    gather(j, bj, gsems[bj]).start()

                gather(i, b, gsems[b]).wait()

                @plsc.parallel_loop(0, chunk, unroll=4)
                def _(r):
                    for jj in range(D_MODEL // LANES):
                        sl = pl.ds(jj * LANES, LANES)
                        rows_v[b, r, sl] = rows_v[b, r, sl] * SCALE

                scatter(i, b, ssems[b]).start()
            return 0

        lax.fori_loop(0, outer, outer_body, 0)
        for b in range(NBUF):
            scatter(num_chunks - NBUF + b, b, ssems[b]).wait()

    return k(table_c, x_flat)


def kernel(x, table):
    total_rows = x.shape[0] * x.shape[1]
    x_flat = x.reshape(total_rows).astype(jnp.int32) * 2
    vocab = table.shape[0]
    table_p = jnp.pad(table, ((0, 0), (0, D_PAD - D_MODEL)))
    table_v = table_p.reshape(2 * vocab, D_MODEL)
    out = _embed(x_flat, table_v, total_rows=total_rows, chunk=320)
    out = out.reshape(x.shape[0], x.shape[1], D_PAD)[:, :, :D_MODEL]
    return out


# speedup vs baseline: 1.0031x; 1.0031x over previous
import functools
import math

import jax
import jax.numpy as jnp
from jax import lax
from jax.experimental import pallas as pl
from jax.experimental.pallas import tpu as pltpu
from jax.experimental.pallas import tpu_sc as plsc

D_MODEL = 64
D_PAD = 128
SCALE = math.sqrt(D_MODEL)

NUM_CORES = 2
NUM_SUBCORES = 16
NUM_WORKERS = NUM_CORES * NUM_SUBCORES
LANES = 16
NBUF = 4
LOOKAHEAD = 2


@functools.partial(jax.jit, static_argnames=("total_rows", "chunk"))
def _embed(x_flat, table_c, *, total_rows, chunk):
    rows_per_worker = total_rows // NUM_WORKERS
    num_chunks = rows_per_worker // chunk
    assert num_chunks % NBUF == 0
    outer = num_chunks // NBUF
    mesh = plsc.VectorSubcoreMesh(core_axis_name="c", subcore_axis_name="s")

    @functools.partial(
        pl.kernel,
        mesh=mesh,
        out_type=jax.ShapeDtypeStruct((total_rows, D_PAD), jnp.float32),
        scratch_types=[
            pltpu.VMEM((rows_per_worker,), jnp.int32),
            pltpu.VMEM((NBUF, chunk, D_MODEL), jnp.float32),
            [pltpu.SemaphoreType.DMA] * NBUF,
            [pltpu.SemaphoreType.DMA] * NBUF,
        ],
        compiler_params=pltpu.CompilerParams(use_tc_tiling_on_sc=False),
    )
    def k(table_hbm, idx_hbm, out_hbm, idx_v, rows_v, gsems, ssems):
        wid = lax.axis_index("s") * NUM_CORES + lax.axis_index("c")
        base = wid * rows_per_worker
        pltpu.sync_copy(idx_hbm.at[pl.ds(base, rows_per_worker)], idx_v)

        def gather(i, b, sem):
            idx_sl = idx_v.at[pl.ds(i * chunk, chunk)]
            return pltpu.make_async_copy(table_hbm.at[idx_sl], rows_v.at[b], sem)

        def scatter(i, b, sem):
            dst = out_hbm.at[pl.ds(base + i * chunk, chunk), pl.ds(0, D_MODEL)]
            return pltpu.make_async_copy(rows_v.at[b], dst, sem)

        for b in range(LOOKAHEAD):
            gather(b, b, gsems[b]).start()

        def outer_body(t, _):
            for b in range(NBUF):
                i = t * NBUF + b
                j = i + LOOKAHEAD
                bj = (b + LOOKAHEAD) % NBUF

                @pl.when(j < num_chunks)
                def _():
                    @pl.when(j >= NBUF)
                    def _():
                        scatter(j - NBUF, bj, ssems[bj]).wait()

                    gather(j, bj, gsems[bj]).start()

                gather(i, b, gsems[b]).wait()

                @plsc.parallel_loop(0, chunk, unroll=4)
                def _(r):
                    for jj in range(D_MODEL // LANES):
                        sl = pl.ds(jj * LANES, LANES)
                        rows_v[b, r, sl] = rows_v[b, r, sl] * SCALE

                scatter(i, b, ssems[b]).start()
            return 0

        lax.fori_loop(0, outer, outer_body, 0)
        for b in range(NBUF):
            scatter(num_chunks - NBUF + b, b, ssems[b]).wait()

    return k(table_c, x_flat)


def kernel(x, table):
    total_rows = x.shape[0] * x.shape[1]
    x_flat = x.reshape(total_rows).astype(jnp.int32) * 2
    vocab = table.shape[0]
    table_p = jnp.pad(table, ((0, 0), (0, D_PAD - D_MODEL)))
    table_v = table_p.reshape(2 * vocab, D_MODEL)
    out = _embed(x_flat, table_v, total_rows=total_rows, chunk=400)
    out = out.reshape(x.shape[0], x.shape[1], D_PAD)[:, :, :D_MODEL]
    return out
